# baseline (speedup 1.0000x reference)
import jax
import jax.numpy as jnp
from jax.experimental import pallas as pl
from jax.experimental.pallas import tpu as pltpu

_MAX_DEPTH = 3
_B = 4
_S = 8192
_H = 1024
_K = max(1, int(_S * (1.0 / _MAX_DEPTH)))
_BLK = 2048
_N = _B * _S


def _router_kernel(hs0_ref, hs1_ref, theta_ref, depth_ref, mem_ref, loss_ref,
                   ones_ref, scores_vmem, acc_vmem):
    step = pl.program_id(0)
    nsteps = pl.num_programs(0)

    th = theta_ref[...]
    b = step // (_S // _BLK)
    s_off = (step % (_S // _BLK)) * _BLK
    part = jnp.zeros((_MAX_DEPTH, 1), jnp.float32)
    for half, ref in enumerate((hs0_ref, hs1_ref)):
        blk = ref[0]
        logits = jax.lax.dot_general(
            th, blk, (((1,), (1,)), ((), ())),
            preferred_element_type=jnp.float32)
        scores = jax.nn.sigmoid(logits)
        scores_vmem[pl.ds(b + 2 * half, 1), pl.ds(s_off, _BLK)] = \
            scores[0:1, :]
        part += jnp.sum(jax.nn.sigmoid(scores), axis=1, keepdims=True)

    @pl.when(step == 0)
    def _():
        acc_vmem[...] = jnp.zeros_like(acc_vmem)

    acc_vmem[...] += part

    @pl.when(step == nsteps - 1)
    def _():
        sc = scores_vmem[...]
        bits = jax.lax.bitcast_convert_type(sc, jnp.int32)

        def count_ge(t):
            return jnp.sum((bits >= t).astype(jnp.int32), axis=1,
                           keepdims=True)

        cur = jnp.zeros((_B, 1), jnp.int32)

        def vstep(i, cur):
            hi = jnp.int32(1) << (29 - 2 * i)
            lo = jnp.int32(1) << (28 - 2 * i)
            c01 = cur | lo
            c10 = cur | hi
            c11 = cur | hi | lo
            ok01 = count_ge(c01) >= _K
            ok10 = count_ge(c10) >= _K
            ok11 = count_ge(c11) >= _K
            return jnp.where(ok11, c11,
                             jnp.where(ok10, c10,
                                       jnp.where(ok01, c01, cur)))

        v = jax.lax.fori_loop(0, 15, vstep, cur)

        cnt_gt = jnp.sum((bits > v).astype(jnp.int32), axis=1, keepdims=True)
        need = _K - cnt_gt
        eq = (bits == v)
        eqi = eq.astype(jnp.int32)
        idx = jax.lax.broadcasted_iota(jnp.int32, (_B, _S), 1)

        def mstep(i, cur):
            bit = jnp.int32(1) << (12 - i)
            cand = cur + bit
            cnt = jnp.sum(jnp.where(idx <= cand - 1, eqi, 0), axis=1,
                          keepdims=True)
            return jnp.where(cnt < need, cand, cur)

        m = jax.lax.fori_loop(0, 13, mstep, jnp.zeros((_B, 1), jnp.int32))
        mem = (bits > v) | (eq & (idx <= m))
        depth_ref[...] = jnp.where(mem, 3, 1).astype(jnp.int32)
        mem_ref[...] = mem
        ones_ref[...] = jnp.full((_B, _S), True, jnp.bool_)

        probs = acc_vmem[...] / jnp.float32(_N)
        target = jnp.float32(1.0 / _MAX_DEPTH)
        loss = jnp.sum(target * (jnp.log(target) - jnp.log(probs)))
        loss_ref[...] = jnp.full((1, 1), loss / _MAX_DEPTH, jnp.float32)


@jax.jit
def kernel(hidden_states, theta):
    hs = hidden_states.reshape(2, _N // 2, _H)
    grid = _N // 2 // _BLK
    depth, mem, loss, mask0 = pl.pallas_call(
        _router_kernel,
        grid=(grid,),
        in_specs=[
            pl.BlockSpec((1, _BLK, _H), lambda i: (0, i, 0)),
            pl.BlockSpec((1, _BLK, _H), lambda i: (1, i, 0)),
            pl.BlockSpec((_MAX_DEPTH, _H), lambda i: (0, 0)),
        ],
        out_specs=[
            pl.BlockSpec((_B, _S), lambda i: (0, 0)),
            pl.BlockSpec((_B, _S), lambda i: (0, 0)),
            pl.BlockSpec((1, 1), lambda i: (0, 0)),
            pl.BlockSpec((_B, _S), lambda i: (0, 0)),
        ],
        out_shape=[
            jax.ShapeDtypeStruct((_B, _S), jnp.int32),
            jax.ShapeDtypeStruct((_B, _S), jnp.bool_),
            jax.ShapeDtypeStruct((1, 1), jnp.float32),
            jax.ShapeDtypeStruct((_B, _S), jnp.bool_),
        ],
        scratch_shapes=[
            pltpu.VMEM((_B, _S), jnp.float32),
            pltpu.VMEM((_MAX_DEPTH, 1), jnp.float32),
        ],
    )(hs, hs, theta)
    return depth, loss.reshape(()), mask0, mem, mem


# speedup vs baseline: 1.0282x; 1.0282x over previous
import jax
import jax.numpy as jnp
from jax.experimental import pallas as pl
from jax.experimental.pallas import tpu as pltpu

_MAX_DEPTH = 3
_B = 4
_S = 8192
_H = 1024
_K = max(1, int(_S * (1.0 / _MAX_DEPTH)))
_BLK = 2048
_N = _B * _S


def _router_kernel(hs_ref, theta_ref, depth_ref, mem_ref, loss_ref,
                   ones_ref, scores_vmem, acc_vmem):
    step = pl.program_id(0)
    nsteps = pl.num_programs(0)

    blk = hs_ref[...]
    th = theta_ref[...]
    logits = jax.lax.dot_general(
        th, blk, (((1,), (1,)), ((), ())),
        preferred_element_type=jnp.float32)
    scores = jax.nn.sigmoid(logits)
    b = step // (_S // _BLK)
    s_off = (step % (_S // _BLK)) * _BLK
    scores_vmem[pl.ds(b, 1), pl.ds(s_off, _BLK)] = scores[0:1, :]

    @pl.when(step == 0)
    def _():
        acc_vmem[...] = jnp.zeros_like(acc_vmem)

    part = jnp.sum(jax.nn.sigmoid(scores), axis=1, keepdims=True)
    acc_vmem[...] += part

    @pl.when(step == nsteps - 1)
    def _():
        sc = scores_vmem[...]
        bits = jax.lax.bitcast_convert_type(sc, jnp.int32)

        def count_ge(t):
            return jnp.sum((bits >= t).astype(jnp.int32), axis=1,
                           keepdims=True)

        cur = jnp.zeros((_B, 1), jnp.int32)

        def vstep(i, cur):
            hi = jnp.int32(1) << (29 - 2 * i)
            lo = jnp.int32(1) << (28 - 2 * i)
            c01 = cur | lo
            c10 = cur | hi
            c11 = cur | hi | lo
            ok01 = count_ge(c01) >= _K
            ok10 = count_ge(c10) >= _K
            ok11 = count_ge(c11) >= _K
            return jnp.where(ok11, c11,
                             jnp.where(ok10, c10,
                                       jnp.where(ok01, c01, cur)))

        v = jax.lax.fori_loop(0, 15, vstep, cur)

        cnt_gt = jnp.sum((bits > v).astype(jnp.int32), axis=1, keepdims=True)
        need = _K - cnt_gt
        eq = (bits == v)
        eqi = eq.astype(jnp.int32)
        idx = jax.lax.broadcasted_iota(jnp.int32, (_B, _S), 1)

        def mstep(i, cur):
            bit = jnp.int32(1) << (12 - i)
            cand = cur + bit
            cnt = jnp.sum(jnp.where(idx <= cand - 1, eqi, 0), axis=1,
                          keepdims=True)
            return jnp.where(cnt < need, cand, cur)

        m = jax.lax.fori_loop(0, 13, mstep, jnp.zeros((_B, 1), jnp.int32))
        mem = (bits > v) | (eq & (idx <= m))
        depth_ref[...] = jnp.where(mem, 3, 1).astype(jnp.int32)
        mem_ref[...] = mem
        ones_ref[...] = jnp.full((_B, _S), True, jnp.bool_)

        probs = acc_vmem[...] / jnp.float32(_N)
        target = jnp.float32(1.0 / _MAX_DEPTH)
        loss = jnp.sum(target * (jnp.log(target) - jnp.log(probs)))
        loss_ref[...] = jnp.full((1, 1), loss / _MAX_DEPTH, jnp.float32)


@jax.jit
def kernel(hidden_states, theta):
    hs = hidden_states.reshape(_N, _H)
    grid = _N // _BLK
    depth, mem, loss, mask0 = pl.pallas_call(
        _router_kernel,
        grid=(grid,),
        in_specs=[
            pl.BlockSpec((_BLK, _H), lambda i: (i, 0)),
            pl.BlockSpec((_MAX_DEPTH, _H), lambda i: (0, 0)),
        ],
        out_specs=[
            pl.BlockSpec((_B, _S), lambda i: (0, 0)),
            pl.BlockSpec((_B, _S), lambda i: (0, 0)),
            pl.BlockSpec((1, 1), lambda i: (0, 0)),
            pl.BlockSpec((_B, _S), lambda i: (0, 0)),
        ],
        out_shape=[
            jax.ShapeDtypeStruct((_B, _S), jnp.int32),
            jax.ShapeDtypeStruct((_B, _S), jnp.bool_),
            jax.ShapeDtypeStruct((1, 1), jnp.float32),
            jax.ShapeDtypeStruct((_B, _S), jnp.bool_),
        ],
        scratch_shapes=[
            pltpu.VMEM((_B, _S), jnp.float32),
            pltpu.VMEM((_MAX_DEPTH, 1), jnp.float32),
        ],
    )(hs, theta)
    return depth, loss.reshape(()), mask0, mem, mem
